# baseline (speedup 1.0000x reference)
import jax
import jax.numpy as jnp
from jax import lax
from jax.experimental import pallas as pl
from jax.experimental.pallas import tpu as pltpu
from jax.experimental.pallas import tpu_sc as plsc

N_ROWS = 16384
N_COLS = 4096
NW = 32
CPW = N_COLS // NW
RCH = 256
NCHUNK = N_ROWS // RCH
K = (N_ROWS - 1) // 2
I32_MAX = 2147483647


def _sc_body(x_hbm, out_hbm, buf, hist, res):
    wid = lax.axis_index("s") * 2 + lax.axis_index("c")
    col0 = wid * CPW
    iota16 = lax.iota(jnp.int32, 16)
    ones16 = jnp.ones((16,), jnp.int32)

    z16 = jnp.zeros((16,), jnp.int32)
    prefs = [z16 for _ in range(8)]
    krems = [jnp.full((16,), K, jnp.int32) for _ in range(8)]

    for p in range(4):
        dshift = 24 - 8 * p
        pshift = 32 - 8 * p
        c_mask = (1 << (8 * p)) - 1

        def zh(i, _):
            hist[pl.ds(i * 16, 16)] = jnp.zeros((16,), jnp.int32)
            return 0

        lax.fori_loop(0, 2048, zh, 0, unroll=8)

        def chunk(ci, _, p=p, dshift=dshift, pshift=pshift,
                  c_mask=c_mask, prefs=prefs):
            pltpu.sync_copy(
                x_hbm.at[pl.ds(ci * RCH, RCH), pl.ds(col0, CPW)], buf)

            def rowb(ri, __):
                for j in range(8):
                    v = buf[ri, pl.ds(j * 16, 16)]
                    s = lax.bitcast_convert_type(v, jnp.int32)
                    ku = s ^ ((s >> 31) | (-2147483648))
                    dig = (ku >> dshift) & 255
                    idx = (dig << 7) + (iota16 + (j * 16))
                    if p == 0:
                        plsc.addupdate_scatter(hist, [idx], ones16)
                    else:
                        kt = (ku >> pshift) & c_mask
                        one_if = jnp.where(kt == prefs[j], 1, 0)
                        plsc.addupdate_scatter(hist, [idx], one_if)
                return 0

            lax.fori_loop(0, RCH, rowb, 0, unroll=2)
            return 0

        lax.fori_loop(0, NCHUNK, chunk, 0)

        for j in range(8):
            kr = krems[j]

            def sbody(d, carry, kr=kr, j=j):
                acc, dig, below, found = carry
                c = hist[pl.ds(d * 128 + j * 16, 16)]
                na = acc + c
                cond = jnp.logical_and(found == 0, na > kr)
                dig = jnp.where(cond, d, dig)
                below = jnp.where(cond, acc, below)
                found = jnp.where(cond, 1, found)
                return (na, dig, below, found)

            _, dig, below, _ = lax.fori_loop(0, 256, sbody, (z16, z16, z16, z16))
            krems[j] = kr - below
            prefs[j] = (prefs[j] << 8) | dig

    keylos = [pf ^ (-2147483648) for pf in prefs]
    m16 = jnp.full((16,), I32_MAX, jnp.int32)

    def chunk2(ci, carry):
        cnts, mns = carry
        pltpu.sync_copy(
            x_hbm.at[pl.ds(ci * RCH, RCH), pl.ds(col0, CPW)], buf)

        def rowb2(ri, cr):
            cs, ms = cr
            ncs = []
            nms = []
            for j in range(8):
                v = buf[ri, pl.ds(j * 16, 16)]
                s = lax.bitcast_convert_type(v, jnp.int32)
                key = s ^ ((s >> 31) & 0x7FFFFFFF)
                ncs.append(cs[j] + jnp.where(key <= keylos[j], 1, 0))
                nms.append(jnp.minimum(
                    ms[j], jnp.where(key > keylos[j], key, I32_MAX)))
            return (tuple(ncs), tuple(nms))

        return lax.fori_loop(0, RCH, rowb2, (cnts, mns), unroll=2)

    cnts, mns = lax.fori_loop(
        0, NCHUNK, chunk2, (tuple([z16] * 8), tuple([m16] * 8)))

    def fromkey(k):
        s = k ^ ((k >> 31) & 0x7FFFFFFF)
        return lax.bitcast_convert_type(s, jnp.float32)

    for j in range(8):
        keylo = keylos[j]
        keyhi = jnp.where(cnts[j] >= K + 2, keylo, mns[j])
        lower = fromkey(keylo)
        upper = fromkey(keyhi)
        res[pl.ds(j * 16, 16)] = jnp.abs(lower - (lower + upper) * 0.5)

    pltpu.sync_copy(res, out_hbm.at[pl.ds(col0, CPW)])


@jax.jit
def kernel(x):
    run = pl.kernel(
        _sc_body,
        out_type=jax.ShapeDtypeStruct((N_COLS,), jnp.float32),
        mesh=plsc.VectorSubcoreMesh(core_axis_name="c", subcore_axis_name="s"),
        compiler_params=pltpu.CompilerParams(needs_layout_passes=False),
        scratch_types=[
            pltpu.VMEM((RCH, CPW), jnp.float32),
            pltpu.VMEM((256 * CPW,), jnp.int32),
            pltpu.VMEM((CPW,), jnp.float32),
        ],
    )
    return run(x)


# speedup vs baseline: 4.7097x; 4.7097x over previous
import jax
import jax.numpy as jnp
from jax import lax
from jax.experimental import pallas as pl
from jax.experimental.pallas import tpu as pltpu
from jax.experimental.pallas import tpu_sc as plsc

N_ROWS = 16384
N_COLS = 4096
K = (N_ROWS - 1) // 2
I32_MAX = 2147483647

SC_COLS = 512
TC_COLS = N_COLS - SC_COLS

TILE_C = 128


def _to_key(f):
    s = jax.lax.bitcast_convert_type(f, jnp.int32)
    return s ^ ((s >> 31) & 0x7FFFFFFF)


def _from_key(k):
    s = k ^ ((k >> 31) & 0x7FFFFFFF)
    return jax.lax.bitcast_convert_type(s, jnp.float32)


def _tc_body(x_ref, o_ref):
    key = _to_key(x_ref[...])
    ktop = (key >> 16).astype(jnp.int16)
    klow = ((key & 0xFFFF) ^ 0x8000).astype(jnp.int16)

    ones_row = jnp.ones((1, N_ROWS), dtype=jnp.bfloat16)

    def count_below(vals, q):
        mask = jnp.where(vals < q, jnp.bfloat16(1), jnp.bfloat16(0))
        return jnp.dot(ones_row, mask, preferred_element_type=jnp.float32)

    kf = jnp.float32(K)

    def step_a(i, p):
        bit = jax.lax.shift_left(jnp.ones((), jnp.int32), 15 - i)
        q = p + bit
        return jnp.where(count_below(ktop, q.astype(jnp.int16)) <= kf, q, p)

    p16 = jax.lax.fori_loop(
        0, 16, step_a, jnp.full((1, TILE_C), -32768, dtype=jnp.int32))

    p16_16 = p16.astype(jnp.int16)
    c0 = count_below(ktop, p16_16)
    mlow = jnp.where(ktop == p16_16, klow, jnp.int16(32767))

    kb = kf - c0

    def step_b(i, p):
        bit = jax.lax.shift_left(jnp.ones((), jnp.int32), 15 - i)
        q = p + bit
        return jnp.where(count_below(mlow, q.astype(jnp.int16)) <= kb, q, p)

    plow = jax.lax.fori_loop(
        0, 16, step_b, jnp.full((1, TILE_C), -32768, dtype=jnp.int32))

    key_lo = (p16 << 16) | ((plow & 0xFFFF) ^ 0x8000)

    le_mask = jnp.where(key <= key_lo, 1.0, 0.0).astype(jnp.bfloat16)
    cnt_le = jnp.dot(ones_row, le_mask, preferred_element_type=jnp.float32)
    above = jnp.where(key > key_lo, key, I32_MAX)
    key_hi = jnp.where(cnt_le >= jnp.float32(K + 2), key_lo,
                       jnp.min(above, axis=0, keepdims=True))

    lower = _from_key(key_lo)
    upper = _from_key(key_hi)
    o_ref[...] = jnp.abs(lower - (lower + upper) * 0.5)


def _tc_part(x):
    out2d = pl.pallas_call(
        _tc_body,
        grid=(TC_COLS // TILE_C,),
        in_specs=[pl.BlockSpec((N_ROWS, TILE_C), lambda i: (0, i))],
        out_specs=pl.BlockSpec((1, TILE_C), lambda i: (0, i)),
        out_shape=jax.ShapeDtypeStruct((1, TC_COLS), jnp.float32),
    )(x)
    return out2d[0]


NW = 32
RCH = 512
NCHUNK = N_ROWS // RCH
SC_BASE = TC_COLS


def _sc_body(x_hbm, out_hbm, buf, hist, res):
    wid = lax.axis_index("s") * 2 + lax.axis_index("c")
    blk = wid // 8
    g0 = (wid % 8) * 16
    col0 = SC_BASE + blk * 128
    iota16 = lax.iota(jnp.int32, 16)
    ones16 = jnp.ones((16,), jnp.int32)

    z16 = jnp.zeros((16,), jnp.int32)
    pref = z16
    krem = jnp.full((16,), K, jnp.int32)

    for p in range(4):
        dshift = 24 - 8 * p
        pshift = 32 - 8 * p
        c_mask = (1 << (8 * p)) - 1

        def zh(i, _):
            hist[pl.ds(i * 16, 16)] = jnp.zeros((16,), jnp.int32)
            return 0

        lax.fori_loop(0, 256, zh, 0, unroll=8)

        def chunk(ci, _, p=p, dshift=dshift, pshift=pshift,
                  c_mask=c_mask, pref=pref):
            pltpu.sync_copy(
                x_hbm.at[pl.ds(ci * RCH, RCH), pl.ds(col0, 128)], buf)

            def rowb(ri, __):
                v = buf[ri, pl.ds(g0, 16)]
                s = lax.bitcast_convert_type(v, jnp.int32)
                ku = s ^ ((s >> 31) | (-2147483648))
                dig = (ku >> dshift) & 255
                idx = (dig << 4) + iota16
                if p == 0:
                    plsc.addupdate_scatter(hist, [idx], ones16)
                else:
                    kt = (ku >> pshift) & c_mask
                    one_if = jnp.where(kt == pref, 1, 0)
                    plsc.addupdate_scatter(hist, [idx], one_if)
                return 0

            lax.fori_loop(0, RCH, rowb, 0, unroll=4)
            return 0

        lax.fori_loop(0, NCHUNK, chunk, 0)

        def sbody(d, carry):
            acc, dig, below, found = carry
            c = hist[pl.ds(d * 16, 16)]
            na = acc + c
            cond = jnp.logical_and(found == 0, na > krem)
            dig = jnp.where(cond, d, dig)
            below = jnp.where(cond, acc, below)
            found = jnp.where(cond, 1, found)
            return (na, dig, below, found)

        _, dig, below, _ = lax.fori_loop(0, 256, sbody, (z16, z16, z16, z16))
        krem = krem - below
        pref = (pref << 8) | dig

    keylo = pref ^ (-2147483648)
    m16 = jnp.full((16,), I32_MAX, jnp.int32)

    def chunk2(ci, carry):
        cnt, mn = carry
        pltpu.sync_copy(
            x_hbm.at[pl.ds(ci * RCH, RCH), pl.ds(col0, 128)], buf)

        def rowb2(ri, cr):
            c, m = cr
            v = buf[ri, pl.ds(g0, 16)]
            s = lax.bitcast_convert_type(v, jnp.int32)
            key = s ^ ((s >> 31) & 0x7FFFFFFF)
            c = c + jnp.where(key <= keylo, 1, 0)
            m = jnp.minimum(m, jnp.where(key > keylo, key, I32_MAX))
            return (c, m)

        return lax.fori_loop(0, RCH, rowb2, (cnt, mn), unroll=4)

    cnt, mn = lax.fori_loop(0, NCHUNK, chunk2, (z16, m16))

    def fromkey(k):
        s = k ^ ((k >> 31) & 0x7FFFFFFF)
        return lax.bitcast_convert_type(s, jnp.float32)

    keyhi = jnp.where(cnt >= K + 2, keylo, mn)
    lower = fromkey(keylo)
    upper = fromkey(keyhi)
    res[pl.ds(0, 16)] = jnp.abs(lower - (lower + upper) * 0.5)

    pltpu.sync_copy(res, out_hbm.at[pl.ds(wid * 16, 16)])


def _sc_part(x):
    run = pl.kernel(
        _sc_body,
        out_type=jax.ShapeDtypeStruct((SC_COLS,), jnp.float32),
        mesh=plsc.VectorSubcoreMesh(core_axis_name="c", subcore_axis_name="s"),
        compiler_params=pltpu.CompilerParams(needs_layout_passes=False),
        scratch_types=[
            pltpu.VMEM((RCH, 128), jnp.float32),
            pltpu.VMEM((256 * 16,), jnp.int32),
            pltpu.VMEM((16,), jnp.float32),
        ],
    )
    return run(x)


@jax.jit
def kernel(x):
    sc_out = _sc_part(x)
    tc_out = _tc_part(x)
    return jnp.concatenate([tc_out, sc_out])


# speedup vs baseline: 6.2484x; 1.3267x over previous
import jax
import jax.numpy as jnp
from jax.experimental import pallas as pl

N_ROWS = 16384
N_COLS = 4096
TILE_C = 128
K = (N_ROWS - 1) // 2


def _to_key(f):
    s = jax.lax.bitcast_convert_type(f, jnp.int32)
    return s ^ ((s >> 31) & 0x7FFFFFFF)


def _from_key(k):
    s = k ^ ((k >> 31) & 0x7FFFFFFF)
    return jax.lax.bitcast_convert_type(s, jnp.float32)


def _median_pair_body(x_ref, o_ref):
    key = _to_key(x_ref[...])
    ktop = (key >> 16).astype(jnp.int16)
    klow = ((key & 0xFFFF) ^ 0x8000).astype(jnp.int16)

    ones_row = jnp.ones((1, N_ROWS), dtype=jnp.bfloat16)

    def count_below(vals, q):
        mask = jnp.where(vals < q, jnp.bfloat16(1), jnp.bfloat16(0))
        return jnp.dot(ones_row, mask, preferred_element_type=jnp.float32)

    kf = jnp.float32(K)

    def step_a(i, p):
        bit = jax.lax.shift_left(jnp.ones((), jnp.int32), 15 - i)
        q = p + bit
        return jnp.where(count_below(ktop, q.astype(jnp.int16)) <= kf, q, p)

    p16 = jax.lax.fori_loop(
        0, 16, step_a, jnp.full((1, TILE_C), -32768, dtype=jnp.int32))

    p16_16 = p16.astype(jnp.int16)
    c0 = count_below(ktop, p16_16)
    mlow = jnp.where(ktop == p16_16, klow, jnp.int16(32767))

    kb = kf - c0

    def step_b(i, p):
        bit = jax.lax.shift_left(jnp.ones((), jnp.int32), 15 - i)
        q = p + bit
        return jnp.where(count_below(mlow, q.astype(jnp.int16)) <= kb, q, p)

    plow = jax.lax.fori_loop(
        0, 16, step_b, jnp.full((1, TILE_C), -32768, dtype=jnp.int32))

    key_lo = (p16 << 16) | ((plow & 0xFFFF) ^ 0x8000)

    le_mask = jnp.where(key <= key_lo, 1.0, 0.0).astype(jnp.bfloat16)
    cnt_le = jnp.dot(ones_row, le_mask, preferred_element_type=jnp.float32)
    above = jnp.where(key > key_lo, key, 2147483647)
    key_hi = jnp.where(cnt_le >= jnp.float32(K + 2), key_lo,
                       jnp.min(above, axis=0, keepdims=True))

    lower = _from_key(key_lo)
    upper = _from_key(key_hi)
    o_ref[...] = jnp.abs(lower - (lower + upper) * 0.5)


@jax.jit
def kernel(x):
    out2d = pl.pallas_call(
        _median_pair_body,
        grid=(N_COLS // TILE_C,),
        in_specs=[pl.BlockSpec((N_ROWS, TILE_C), lambda i: (0, i))],
        out_specs=pl.BlockSpec((1, TILE_C), lambda i: (0, i)),
        out_shape=jax.ShapeDtypeStruct((1, N_COLS), jnp.float32),
    )(x)
    return out2d[0]
